# baseline (speedup 1.0000x reference)
import jax
import jax.numpy as jnp
from jax.experimental import pallas as pl
from jax.experimental.pallas import tpu as pltpu

_P = 16
_N = 8
_WS = 64
_NC = _WS - _P + 1
_BC = 16
_NB = 7
_TP = 56 * 64

_INTERPRET = False
_INF = float("inf")


def _dist_body(ridx_ref, wr_ref, wi_ref, rwr_ref, rwi_ref, dist_ref):
    rx = ridx_ref[0]
    ry = ridx_ref[1]
    nx_valid = jnp.minimum(rx, _WS // 2) + _P + 1
    ny_valid = jnp.minimum(ry, _WS // 2) + _P + 1
    iota_j = jax.lax.broadcasted_iota(jnp.int32, (1, _WS), 1)

    def dist_pair(p, _):
        bc0 = p * 2
        bc1 = p * 2 + 1

        def dist_blk(i8, __):
            base = pl.multiple_of(i8 * 8, 8)
            wbr = jnp.concatenate(
                [wr_ref[bc0, pl.ds(base, 24), :],
                 wr_ref[bc1, pl.ds(base, 24), :]], axis=1)
            wbi = jnp.concatenate(
                [wi_ref[bc0, pl.ds(base, 24), :],
                 wi_ref[bc1, pl.ds(base, 24), :]], axis=1)
            rows = []
            for r in range(8):
                acc = jnp.zeros((_P, 113), jnp.float32)
                for v in range(_P):
                    dr_ = wbr[r:r + _P, v:v + 113] - rwr_ref[p, v, :, :113]
                    di_ = wbi[r:r + _P, v:v + 113] - rwi_ref[p, v, :, :113]
                    acc = acc + (dr_ * dr_ + di_ * di_)
                row = jnp.sum(acc, axis=0, keepdims=True)
                row = (row * (1.0 / (_P * _P))) * 0.5
                ok = ((i8 * 8 + r) < nx_valid) & (iota_j < ny_valid)
                row0 = jnp.concatenate(
                    [row[:, :_NC], jnp.full((1, _WS - _NC), _INF)], axis=1)
                row1 = jnp.concatenate(
                    [row[:, 64:113], jnp.full((1, _WS - _NC), _INF)], axis=1)
                rows.append(jnp.where(ok, row0, _INF))
                rows.append(jnp.where(ok, row1, _INF))
            blk0 = jnp.concatenate(rows[0::2], axis=0)
            blk1 = jnp.concatenate(rows[1::2], axis=0)
            dist_ref[bc0, pl.ds(base, 8), :] = blk0
            dist_ref[bc1, pl.ds(base, 8), :] = blk1
            return 0

        jax.lax.fori_loop(0, _NB, dist_blk, 0, unroll=False)
        return 0

    jax.lax.fori_loop(0, _BC // 2, dist_pair, 0, unroll=False)


def _scan_body(distt_ref, ni_ref):
    iota_s = jax.lax.broadcasted_iota(jnp.int32, (_N, _BC), 0)

    def scan_blk(b8, carry):
        nd, ni, ndmax = carry
        blk = distt_ref[b8]

        def insert(c2):
            nd2, ni2, _ = c2
            for r in range(8):
                db = jnp.broadcast_to(blk[r:r + 1, :], (_N, _BC))
                mask = nd2 > db
                idx = jnp.min(jnp.where(mask, iota_s, _N), axis=0,
                              keepdims=True)
                sel = (iota_s == idx)
                nd2 = jnp.where(sel, db, nd2)
                ni2 = jnp.where(sel, b8 * 8 + r, ni2)
            return nd2, ni2, jnp.max(nd2, axis=0, keepdims=True)

        can = jnp.any(blk < jnp.broadcast_to(ndmax, (_N, _BC)))
        return jax.lax.cond(can, insert, lambda c2: c2, (nd, ni, ndmax))

    nd0 = jnp.full((_N, _BC), 1e33, jnp.float32)
    ni0 = jnp.full((_N, _BC), -1, jnp.int32)
    nm0 = jnp.full((1, _BC), 1e33, jnp.float32)
    _, ni, _ = jax.lax.fori_loop(0, _TP // 8, scan_blk, (nd0, ni0, nm0))
    ni_ref[...] = ni


def _gather_body(ni_ref, w2_ref, out_ref):
    iota_ex = (jax.lax.broadcasted_iota(jnp.int32, (_P, _WS), 1)
               - jax.lax.broadcasted_iota(jnp.int32, (_P, _WS), 0))
    iota_fy = (jax.lax.broadcasted_iota(jnp.int32, (2 * _WS, 2 * _P), 0)
               - jax.lax.broadcasted_iota(jnp.int32, (2 * _WS, 2 * _P), 1))
    for bc in range(_BC):
        wv = w2_ref[bc]
        for s in range(_N):
            t = ni_ref[s, bc]
            ok = t >= 0
            tt = jnp.maximum(t, 0)
            i = tt // _WS
            j = tt - i * _WS
            ei = (iota_ex == i).astype(jnp.float32)
            fj = (iota_fy == 2 * j).astype(jnp.float32)
            tmp = jnp.dot(wv, fj, preferred_element_type=jnp.float32,
                          precision=jax.lax.Precision.HIGHEST)
            patch = jnp.dot(ei, tmp, preferred_element_type=jnp.float32,
                            precision=jax.lax.Precision.HIGHEST)
            out_ref[bc, s] = jnp.where(ok, patch, 0.0)


def kernel(data, reference_index):
    ridx = reference_index.astype(jnp.int32)
    B, C = data.shape[0], data.shape[1]
    rx, ry = ridx[0], ridx[1]

    window = jax.lax.dynamic_slice(
        data, (0, 0, rx - _WS // 2, ry - _WS // 2, 0), (B, C, _WS, _WS, 2))
    rpatch = jax.lax.dynamic_slice(
        data, (0, 0, rx, ry, 0), (B, C, _P, _P, 2))

    wr = window[..., 0].reshape(_BC, _WS, _WS)
    wi = window[..., 1].reshape(_BC, _WS, _WS)
    wr = jnp.pad(wr, ((0, 0), (0, 8), (0, 0)))
    wi = jnp.pad(wi, ((0, 0), (0, 8), (0, 0)))
    rp2 = rpatch.reshape(_BC // 2, 2, _P, _P, 2)
    lane_lo = (jnp.arange(2 * _WS) < _WS)[None, None, None, :]
    rwr = jnp.where(lane_lo,
                    jnp.transpose(rp2[:, 0, :, :, 0], (0, 2, 1))[..., None],
                    jnp.transpose(rp2[:, 1, :, :, 0], (0, 2, 1))[..., None])
    rwi = jnp.where(lane_lo,
                    jnp.transpose(rp2[:, 0, :, :, 1], (0, 2, 1))[..., None],
                    jnp.transpose(rp2[:, 1, :, :, 1], (0, 2, 1))[..., None])

    dist = pl.pallas_call(
        _dist_body,
        grid=(),
        in_specs=[
            pl.BlockSpec(memory_space=pltpu.SMEM),
            pl.BlockSpec(memory_space=pltpu.VMEM),
            pl.BlockSpec(memory_space=pltpu.VMEM),
            pl.BlockSpec(memory_space=pltpu.VMEM),
            pl.BlockSpec(memory_space=pltpu.VMEM),
        ],
        out_specs=pl.BlockSpec(memory_space=pltpu.VMEM),
        out_shape=jax.ShapeDtypeStruct((_BC, 56, _WS), jnp.float32),
        interpret=_INTERPRET,
    )(ridx, wr, wi, rwr, rwi)

    distt = dist.reshape(_BC, _TP).T.reshape(_TP // 8, 8, _BC)

    ni = pl.pallas_call(
        _scan_body,
        grid=(),
        in_specs=[pl.BlockSpec(memory_space=pltpu.VMEM)],
        out_specs=pl.BlockSpec(memory_space=pltpu.VMEM),
        out_shape=jax.ShapeDtypeStruct((_N, _BC), jnp.int32),
        interpret=_INTERPRET,
    )(distt)

    w2 = window.reshape(_BC, _WS, 2 * _WS)
    patches = pl.pallas_call(
        _gather_body,
        grid=(),
        in_specs=[
            pl.BlockSpec(memory_space=pltpu.SMEM),
            pl.BlockSpec(memory_space=pltpu.VMEM),
        ],
        out_specs=pl.BlockSpec(memory_space=pltpu.VMEM),
        out_shape=jax.ShapeDtypeStruct((_BC, _N, _P, 2 * _P), jnp.float32),
        interpret=_INTERPRET,
    )(ni, w2)

    patches = patches.reshape(B, C, _N, _P * _P, 2)
    return jax.lax.complex(patches[..., 0], patches[..., 1])


# speedup vs baseline: 2.3000x; 2.3000x over previous
import jax
import jax.numpy as jnp
from jax.experimental import pallas as pl
from jax.experimental.pallas import tpu as pltpu

_P = 16
_N = 8
_WS = 64
_NC = _WS - _P + 1
_BC = 16
_NB = 7
_TP = 56 * 64

_INTERPRET = False
_INF = float("inf")


def _dist_body(ridx_ref, wr_ref, wi_ref, rwr_ref, rwi_ref, dist_ref):
    rx = ridx_ref[0]
    ry = ridx_ref[1]
    nx_valid = jnp.minimum(rx, _WS // 2) + _P + 1
    ny_valid = jnp.minimum(ry, _WS // 2) + _P + 1
    iota_j = jax.lax.broadcasted_iota(jnp.int32, (1, _WS), 1)

    def dist_bc(bc, _):
        rr = rwr_ref[bc]
        ri = rwi_ref[bc]

        def dist_blk(i8, __):
            base = pl.multiple_of(i8 * 8, 8)
            wbr = wr_ref[bc, pl.ds(base, 24), :]
            wbi = wi_ref[bc, pl.ds(base, 24), :]
            rows = []
            for r in range(8):
                acc = jnp.zeros((_P, _NC), jnp.float32)
                for v in range(_P):
                    dr_ = wbr[r:r + _P, v:v + _NC] - rr[:, v:v + 1]
                    di_ = wbi[r:r + _P, v:v + _NC] - ri[:, v:v + 1]
                    acc = acc + (dr_ * dr_ + di_ * di_)
                row = jnp.sum(acc, axis=0, keepdims=True)
                row = (row * (1.0 / (_P * _P))) * 0.5
                row = jnp.concatenate(
                    [row, jnp.full((1, _WS - _NC), _INF)], axis=1)
                ok = ((i8 * 8 + r) < nx_valid) & (iota_j < ny_valid)
                rows.append(jnp.where(ok, row, _INF))
            dist_ref[bc, pl.ds(base, 8), :] = jnp.concatenate(rows, axis=0)
            return 0

        jax.lax.fori_loop(0, _NB, dist_blk, 0, unroll=False)
        return 0

    jax.lax.fori_loop(0, _BC, dist_bc, 0, unroll=False)


def _scan_body(distt_ref, ni_ref):
    iota_s = jax.lax.broadcasted_iota(jnp.int32, (_N, _BC), 0)

    def scan_blk(b8, carry):
        nd, ni, ndmax = carry
        blk = distt_ref[b8]

        def insert(c2):
            nd2, ni2, _ = c2
            for r in range(8):
                db = jnp.broadcast_to(blk[r:r + 1, :], (_N, _BC))
                mask = nd2 > db
                idx = jnp.min(jnp.where(mask, iota_s, _N), axis=0,
                              keepdims=True)
                sel = (iota_s == idx)
                nd2 = jnp.where(sel, db, nd2)
                ni2 = jnp.where(sel, b8 * 8 + r, ni2)
            return nd2, ni2, jnp.max(nd2, axis=0, keepdims=True)

        can = jnp.any(blk < jnp.broadcast_to(ndmax, (_N, _BC)))
        return jax.lax.cond(can, insert, lambda c2: c2, (nd, ni, ndmax))

    nd0 = jnp.full((_N, _BC), 1e33, jnp.float32)
    ni0 = jnp.full((_N, _BC), -1, jnp.int32)
    nm0 = jnp.full((1, _BC), 1e33, jnp.float32)
    _, ni, _ = jax.lax.fori_loop(0, _TP // 8, scan_blk, (nd0, ni0, nm0))
    ni_ref[...] = ni


def _gather_body(ni_ref, w2_ref, out_ref):
    iota_ex = (jax.lax.broadcasted_iota(jnp.int32, (_P, _WS), 1)
               - jax.lax.broadcasted_iota(jnp.int32, (_P, _WS), 0))
    iota_fy = (jax.lax.broadcasted_iota(jnp.int32, (2 * _WS, 2 * _P), 0)
               - jax.lax.broadcasted_iota(jnp.int32, (2 * _WS, 2 * _P), 1))
    for bc in range(_BC):
        wv = w2_ref[bc]
        for s in range(_N):
            t = ni_ref[s, bc]
            ok = t >= 0
            tt = jnp.maximum(t, 0)
            i = tt // _WS
            j = tt - i * _WS
            ei = (iota_ex == i).astype(jnp.float32)
            fj = (iota_fy == 2 * j).astype(jnp.float32)
            tmp = jnp.dot(wv, fj, preferred_element_type=jnp.float32,
                          precision=jax.lax.Precision.HIGHEST)
            patch = jnp.dot(ei, tmp, preferred_element_type=jnp.float32,
                            precision=jax.lax.Precision.HIGHEST)
            out_ref[bc, s] = jnp.where(ok, patch, 0.0)


def kernel(data, reference_index):
    ridx = reference_index.astype(jnp.int32)
    B, C = data.shape[0], data.shape[1]
    rx, ry = ridx[0], ridx[1]

    window = jax.lax.dynamic_slice(
        data, (0, 0, rx - _WS // 2, ry - _WS // 2, 0), (B, C, _WS, _WS, 2))
    rpatch = jax.lax.dynamic_slice(
        data, (0, 0, rx, ry, 0), (B, C, _P, _P, 2))

    wr = window[..., 0].reshape(_BC, _WS, _WS)
    wi = window[..., 1].reshape(_BC, _WS, _WS)
    wr = jnp.pad(wr, ((0, 0), (0, 8), (0, 0)))
    wi = jnp.pad(wi, ((0, 0), (0, 8), (0, 0)))
    rwr = rpatch[..., 0].reshape(_BC, _P, _P)
    rwi = rpatch[..., 1].reshape(_BC, _P, _P)

    dist = pl.pallas_call(
        _dist_body,
        grid=(),
        in_specs=[
            pl.BlockSpec(memory_space=pltpu.SMEM),
            pl.BlockSpec(memory_space=pltpu.VMEM),
            pl.BlockSpec(memory_space=pltpu.VMEM),
            pl.BlockSpec(memory_space=pltpu.VMEM),
            pl.BlockSpec(memory_space=pltpu.VMEM),
        ],
        out_specs=pl.BlockSpec(memory_space=pltpu.VMEM),
        out_shape=jax.ShapeDtypeStruct((_BC, 56, _WS), jnp.float32),
        interpret=_INTERPRET,
    )(ridx, wr, wi, rwr, rwi)

    distt = dist.reshape(_BC, _TP).T.reshape(_TP // 8, 8, _BC)

    ni = pl.pallas_call(
        _scan_body,
        grid=(),
        in_specs=[pl.BlockSpec(memory_space=pltpu.VMEM)],
        out_specs=pl.BlockSpec(memory_space=pltpu.VMEM),
        out_shape=jax.ShapeDtypeStruct((_N, _BC), jnp.int32),
        interpret=_INTERPRET,
    )(distt)

    w2 = window.reshape(_BC, _WS, 2 * _WS)
    patches = pl.pallas_call(
        _gather_body,
        grid=(),
        in_specs=[
            pl.BlockSpec(memory_space=pltpu.SMEM),
            pl.BlockSpec(memory_space=pltpu.VMEM),
        ],
        out_specs=pl.BlockSpec(memory_space=pltpu.VMEM),
        out_shape=jax.ShapeDtypeStruct((_BC, _N, _P, 2 * _P), jnp.float32),
        interpret=_INTERPRET,
    )(ni, w2)

    patches = patches.reshape(B, C, _N, _P * _P, 2)
    return jax.lax.complex(patches[..., 0], patches[..., 1])


# speedup vs baseline: 2.3008x; 1.0003x over previous
import jax
import jax.numpy as jnp
from jax.experimental import pallas as pl
from jax.experimental.pallas import tpu as pltpu

_P = 16
_N = 8
_WS = 64
_NC = _WS - _P + 1
_BC = 16
_NB = 7
_TP = 56 * 64

_INF = float("inf")


def _dist_body(ridx_ref, wr_ref, wi_ref, rwr_ref, rwi_ref, dist_ref):
    rx = ridx_ref[0]
    ry = ridx_ref[1]
    nx_valid = jnp.minimum(rx, _WS // 2) + _P + 1
    ny_valid = jnp.minimum(ry, _WS // 2) + _P + 1
    iota_j = jax.lax.broadcasted_iota(jnp.int32, (1, _WS), 1)

    def dist_bc(bc, _):
        rr = rwr_ref[bc]
        ri = rwi_ref[bc]

        def dist_blk(i8, __):
            base = pl.multiple_of(i8 * 8, 8)
            wbr = wr_ref[bc, pl.ds(base, 24), :]
            wbi = wi_ref[bc, pl.ds(base, 24), :]
            rows = []
            for r in range(8):
                acc = jnp.zeros((_P, _NC), jnp.float32)
                for v in range(_P):
                    dr_ = wbr[r:r + _P, v:v + _NC] - rr[:, v:v + 1]
                    di_ = wbi[r:r + _P, v:v + _NC] - ri[:, v:v + 1]
                    acc = acc + (dr_ * dr_ + di_ * di_)
                row = jnp.sum(acc, axis=0, keepdims=True)
                row = (row * (1.0 / (_P * _P))) * 0.5
                row = jnp.concatenate(
                    [row, jnp.full((1, _WS - _NC), _INF)], axis=1)
                ok = ((i8 * 8 + r) < nx_valid) & (iota_j < ny_valid)
                rows.append(jnp.where(ok, row, _INF))
            dist_ref[bc, pl.ds(base, 8), :] = jnp.concatenate(rows, axis=0)
            return 0

        jax.lax.fori_loop(0, _NB, dist_blk, 0, unroll=False)
        return 0

    jax.lax.fori_loop(0, _BC, dist_bc, 0, unroll=False)


def _scan_body(distt_ref, ni_ref):
    iota_s = jax.lax.broadcasted_iota(jnp.int32, (_N, _BC), 0)

    def scan_blk(b8, carry):
        nd, ni, ndmax = carry
        blk = distt_ref[b8]

        def insert(c2):
            nd2, ni2, _ = c2
            for r in range(8):
                db = jnp.broadcast_to(blk[r:r + 1, :], (_N, _BC))
                mask = nd2 > db
                idx = jnp.min(jnp.where(mask, iota_s, _N), axis=0,
                              keepdims=True)
                sel = (iota_s == idx)
                nd2 = jnp.where(sel, db, nd2)
                ni2 = jnp.where(sel, b8 * 8 + r, ni2)
            return nd2, ni2, jnp.max(nd2, axis=0, keepdims=True)

        can = jnp.any(blk < jnp.broadcast_to(ndmax, (_N, _BC)))
        return jax.lax.cond(can, insert, lambda c2: c2, (nd, ni, ndmax))

    nd0 = jnp.full((_N, _BC), 1e33, jnp.float32)
    ni0 = jnp.full((_N, _BC), -1, jnp.int32)
    nm0 = jnp.full((1, _BC), 1e33, jnp.float32)
    _, ni, _ = jax.lax.fori_loop(0, _TP // 8, scan_blk, (nd0, ni0, nm0))
    ni_ref[...] = ni


def _gather_body(ni_ref, w2_ref, out_ref):
    iota_ex = (jax.lax.broadcasted_iota(jnp.int32, (_P, _WS), 1)
               - jax.lax.broadcasted_iota(jnp.int32, (_P, _WS), 0))
    iota_fy = (jax.lax.broadcasted_iota(jnp.int32, (2 * _WS, 2 * _P), 0)
               - jax.lax.broadcasted_iota(jnp.int32, (2 * _WS, 2 * _P), 1))
    for bc in range(_BC):
        wv = w2_ref[bc]
        for s in range(_N):
            t = ni_ref[s, bc]
            ok = t >= 0
            tt = jnp.maximum(t, 0)
            i = tt // _WS
            j = tt - i * _WS
            ei = (iota_ex == i).astype(jnp.float32)
            fj = (iota_fy == 2 * j).astype(jnp.float32)
            tmp = jnp.dot(wv, fj, preferred_element_type=jnp.float32,
                          precision=jax.lax.Precision.HIGHEST)
            patch = jnp.dot(ei, tmp, preferred_element_type=jnp.float32,
                            precision=jax.lax.Precision.HIGHEST)
            out_ref[bc, s] = jnp.where(ok, patch, 0.0)


def kernel(data, reference_index):
    ridx = reference_index.astype(jnp.int32)
    B, C = data.shape[0], data.shape[1]
    rx, ry = ridx[0], ridx[1]

    window = jax.lax.dynamic_slice(
        data, (0, 0, rx - _WS // 2, ry - _WS // 2, 0), (B, C, _WS, _WS, 2))
    rpatch = jax.lax.dynamic_slice(
        data, (0, 0, rx, ry, 0), (B, C, _P, _P, 2))

    wr = window[..., 0].reshape(_BC, _WS, _WS)
    wi = window[..., 1].reshape(_BC, _WS, _WS)
    wr = jnp.pad(wr, ((0, 0), (0, 8), (0, 0)))
    wi = jnp.pad(wi, ((0, 0), (0, 8), (0, 0)))
    rwr = rpatch[..., 0].reshape(_BC, _P, _P)
    rwi = rpatch[..., 1].reshape(_BC, _P, _P)

    dist = pl.pallas_call(
        _dist_body,
        grid=(),
        in_specs=[
            pl.BlockSpec(memory_space=pltpu.SMEM),
            pl.BlockSpec(memory_space=pltpu.VMEM),
            pl.BlockSpec(memory_space=pltpu.VMEM),
            pl.BlockSpec(memory_space=pltpu.VMEM),
            pl.BlockSpec(memory_space=pltpu.VMEM),
        ],
        out_specs=pl.BlockSpec(memory_space=pltpu.VMEM),
        out_shape=jax.ShapeDtypeStruct((_BC, 56, _WS), jnp.float32),
    )(ridx, wr, wi, rwr, rwi)

    distt = dist.reshape(_BC, _TP).T.reshape(_TP // 8, 8, _BC)

    ni = pl.pallas_call(
        _scan_body,
        grid=(),
        in_specs=[pl.BlockSpec(memory_space=pltpu.VMEM)],
        out_specs=pl.BlockSpec(memory_space=pltpu.VMEM),
        out_shape=jax.ShapeDtypeStruct((_N, _BC), jnp.int32),
    )(distt)

    w2 = window.reshape(_BC, _WS, 2 * _WS)
    patches = pl.pallas_call(
        _gather_body,
        grid=(),
        in_specs=[
            pl.BlockSpec(memory_space=pltpu.SMEM),
            pl.BlockSpec(memory_space=pltpu.VMEM),
        ],
        out_specs=pl.BlockSpec(memory_space=pltpu.VMEM),
        out_shape=jax.ShapeDtypeStruct((_BC, _N, _P, 2 * _P), jnp.float32),
    )(ni, w2)

    patches = patches.reshape(B, C, _N, _P * _P, 2)
    return jax.lax.complex(patches[..., 0], patches[..., 1])


# speedup vs baseline: 2.3061x; 1.0023x over previous
import jax
import jax.numpy as jnp
from jax.experimental import pallas as pl
from jax.experimental.pallas import tpu as pltpu

_P = 16
_N = 8
_WS = 64
_NC = _WS - _P + 1
_BC = 16
_NB = 7
_TP = 56 * 64

_INF = float("inf")


def _dist_body(ridx_ref, wr_ref, wi_ref, rwr_ref, rwi_ref, dist_ref):
    rx = ridx_ref[0]
    ry = ridx_ref[1]
    nx_valid = jnp.minimum(rx, _WS // 2) + _P + 1
    ny_valid = jnp.minimum(ry, _WS // 2) + _P + 1
    iota_j = jax.lax.broadcasted_iota(jnp.int32, (1, _WS), 1)

    def dist_bc(bc, _):
        rr = rwr_ref[bc]
        ri = rwi_ref[bc]

        def dist_blk(i8, __):
            base = pl.multiple_of(i8 * 8, 8)
            wbr = wr_ref[bc, pl.ds(base, 24), :]
            wbi = wi_ref[bc, pl.ds(base, 24), :]
            rows = []
            for r in range(8):
                acc = jnp.zeros((_P, _NC), jnp.float32)
                for v in range(_P):
                    dr_ = wbr[r:r + _P, v:v + _NC] - rr[:, v:v + 1]
                    di_ = wbi[r:r + _P, v:v + _NC] - ri[:, v:v + 1]
                    acc = acc + (dr_ * dr_ + di_ * di_)
                row = jnp.sum(acc, axis=0, keepdims=True)
                row = (row * (1.0 / (_P * _P))) * 0.5
                row = jnp.concatenate(
                    [row, jnp.full((1, _WS - _NC), _INF)], axis=1)
                ok = ((i8 * 8 + r) < nx_valid) & (iota_j < ny_valid)
                rows.append(jnp.where(ok, row, _INF))
            dist_ref[bc, pl.ds(base, 8), :] = jnp.concatenate(rows, axis=0)
            return 0

        jax.lax.fori_loop(0, _NB, dist_blk, 0, unroll=False)
        return 0

    jax.lax.fori_loop(0, _BC, dist_bc, 0, unroll=2)


def _scan_body(distt_ref, ni_ref):
    iota_s = jax.lax.broadcasted_iota(jnp.int32, (_N, _BC), 0)

    def scan_blk(b8, carry):
        nd, ni, ndmax = carry
        blk = distt_ref[b8]

        def insert(c2):
            nd2, ni2, _ = c2
            for r in range(8):
                db = jnp.broadcast_to(blk[r:r + 1, :], (_N, _BC))
                mask = nd2 > db
                idx = jnp.min(jnp.where(mask, iota_s, _N), axis=0,
                              keepdims=True)
                sel = (iota_s == idx)
                nd2 = jnp.where(sel, db, nd2)
                ni2 = jnp.where(sel, b8 * 8 + r, ni2)
            return nd2, ni2, jnp.max(nd2, axis=0, keepdims=True)

        can = jnp.any(blk < jnp.broadcast_to(ndmax, (_N, _BC)))
        return jax.lax.cond(can, insert, lambda c2: c2, (nd, ni, ndmax))

    nd0 = jnp.full((_N, _BC), 1e33, jnp.float32)
    ni0 = jnp.full((_N, _BC), -1, jnp.int32)
    nm0 = jnp.full((1, _BC), 1e33, jnp.float32)
    _, ni, _ = jax.lax.fori_loop(0, _TP // 8, scan_blk, (nd0, ni0, nm0))
    ni_ref[...] = ni


def _gather_body(ni_ref, w2_ref, out_ref):
    iota_ex = (jax.lax.broadcasted_iota(jnp.int32, (_P, _WS), 1)
               - jax.lax.broadcasted_iota(jnp.int32, (_P, _WS), 0))
    iota_fy = (jax.lax.broadcasted_iota(jnp.int32, (2 * _WS, 2 * _P), 0)
               - jax.lax.broadcasted_iota(jnp.int32, (2 * _WS, 2 * _P), 1))
    for bc in range(_BC):
        wv = w2_ref[bc]
        for s in range(_N):
            t = ni_ref[s, bc]
            ok = t >= 0
            tt = jnp.maximum(t, 0)
            i = tt // _WS
            j = tt - i * _WS
            ei = (iota_ex == i).astype(jnp.float32)
            fj = (iota_fy == 2 * j).astype(jnp.float32)
            tmp = jnp.dot(wv, fj, preferred_element_type=jnp.float32,
                          precision=jax.lax.Precision.HIGHEST)
            patch = jnp.dot(ei, tmp, preferred_element_type=jnp.float32,
                            precision=jax.lax.Precision.HIGHEST)
            out_ref[bc, s] = jnp.where(ok, patch, 0.0)


def kernel(data, reference_index):
    ridx = reference_index.astype(jnp.int32)
    B, C = data.shape[0], data.shape[1]
    rx, ry = ridx[0], ridx[1]

    window = jax.lax.dynamic_slice(
        data, (0, 0, rx - _WS // 2, ry - _WS // 2, 0), (B, C, _WS, _WS, 2))
    rpatch = jax.lax.dynamic_slice(
        data, (0, 0, rx, ry, 0), (B, C, _P, _P, 2))

    wr = window[..., 0].reshape(_BC, _WS, _WS)
    wi = window[..., 1].reshape(_BC, _WS, _WS)
    wr = jnp.pad(wr, ((0, 0), (0, 8), (0, 0)))
    wi = jnp.pad(wi, ((0, 0), (0, 8), (0, 0)))
    rwr = rpatch[..., 0].reshape(_BC, _P, _P)
    rwi = rpatch[..., 1].reshape(_BC, _P, _P)

    dist = pl.pallas_call(
        _dist_body,
        grid=(),
        in_specs=[
            pl.BlockSpec(memory_space=pltpu.SMEM),
            pl.BlockSpec(memory_space=pltpu.VMEM),
            pl.BlockSpec(memory_space=pltpu.VMEM),
            pl.BlockSpec(memory_space=pltpu.VMEM),
            pl.BlockSpec(memory_space=pltpu.VMEM),
        ],
        out_specs=pl.BlockSpec(memory_space=pltpu.VMEM),
        out_shape=jax.ShapeDtypeStruct((_BC, 56, _WS), jnp.float32),
    )(ridx, wr, wi, rwr, rwi)

    distt = dist.reshape(_BC, _TP).T.reshape(_TP // 8, 8, _BC)

    ni = pl.pallas_call(
        _scan_body,
        grid=(),
        in_specs=[pl.BlockSpec(memory_space=pltpu.VMEM)],
        out_specs=pl.BlockSpec(memory_space=pltpu.VMEM),
        out_shape=jax.ShapeDtypeStruct((_N, _BC), jnp.int32),
    )(distt)

    w2 = window.reshape(_BC, _WS, 2 * _WS)
    patches = pl.pallas_call(
        _gather_body,
        grid=(),
        in_specs=[
            pl.BlockSpec(memory_space=pltpu.SMEM),
            pl.BlockSpec(memory_space=pltpu.VMEM),
        ],
        out_specs=pl.BlockSpec(memory_space=pltpu.VMEM),
        out_shape=jax.ShapeDtypeStruct((_BC, _N, _P, 2 * _P), jnp.float32),
    )(ni, w2)

    patches = patches.reshape(B, C, _N, _P * _P, 2)
    return jax.lax.complex(patches[..., 0], patches[..., 1])


# speedup vs baseline: 2.9612x; 1.2841x over previous
import jax
import jax.numpy as jnp
from jax.experimental import pallas as pl
from jax.experimental.pallas import tpu as pltpu

_P = 16
_N = 8
_WS = 64
_NC = _WS - _P + 1
_BC = 16
_NB = 7
_TP = 56 * 64

_INF = float("inf")


def _dist_body(ridx_ref, wr_ref, wi_ref, rwr_ref, rwi_ref, dist_ref):
    rx = ridx_ref[0]
    ry = ridx_ref[1]
    nx_valid = jnp.minimum(rx, _WS // 2) + _P + 1
    ny_valid = jnp.minimum(ry, _WS // 2) + _P + 1
    iota_j = jax.lax.broadcasted_iota(jnp.int32, (1, _WS), 1)

    def dist_bc(bc, _):
        rr = rwr_ref[bc]
        ri = rwi_ref[bc]

        def dist_blk(i8, __):
            base = pl.multiple_of(i8 * 8, 8)
            wbr = wr_ref[bc, pl.ds(base, 24), :]
            wbi = wi_ref[bc, pl.ds(base, 24), :]
            rows = []
            for r in range(8):
                acc = jnp.zeros((_P, _NC), jnp.float32)
                for v in range(_P):
                    dr_ = wbr[r:r + _P, v:v + _NC] - rr[:, v:v + 1]
                    di_ = wbi[r:r + _P, v:v + _NC] - ri[:, v:v + 1]
                    acc = acc + (dr_ * dr_ + di_ * di_)
                row = jnp.sum(acc, axis=0, keepdims=True)
                row = (row * (1.0 / (_P * _P))) * 0.5
                row = jnp.concatenate(
                    [row, jnp.full((1, _WS - _NC), _INF)], axis=1)
                ok = ((i8 * 8 + r) < nx_valid) & (iota_j < ny_valid)
                rows.append(jnp.where(ok, row, _INF))
            dist_ref[bc, pl.ds(base, 8), :] = jnp.concatenate(rows, axis=0)
            return 0

        jax.lax.fori_loop(0, _NB, dist_blk, 0, unroll=7)
        return 0

    jax.lax.fori_loop(0, _BC, dist_bc, 0, unroll=False)


def _scan_body(distt_ref, ni_ref):
    iota_s = jax.lax.broadcasted_iota(jnp.int32, (_N, _BC), 0)

    def scan_blk(b8, carry):
        nd, ni, ndmax = carry
        blk = distt_ref[b8]

        def insert(c2):
            nd2, ni2, _ = c2
            for r in range(8):
                db = jnp.broadcast_to(blk[r:r + 1, :], (_N, _BC))
                mask = nd2 > db
                idx = jnp.min(jnp.where(mask, iota_s, _N), axis=0,
                              keepdims=True)
                sel = (iota_s == idx)
                nd2 = jnp.where(sel, db, nd2)
                ni2 = jnp.where(sel, b8 * 8 + r, ni2)
            return nd2, ni2, jnp.max(nd2, axis=0, keepdims=True)

        can = jnp.any(blk < jnp.broadcast_to(ndmax, (_N, _BC)))
        return jax.lax.cond(can, insert, lambda c2: c2, (nd, ni, ndmax))

    nd0 = jnp.full((_N, _BC), 1e33, jnp.float32)
    ni0 = jnp.full((_N, _BC), -1, jnp.int32)
    nm0 = jnp.full((1, _BC), 1e33, jnp.float32)
    _, ni, _ = jax.lax.fori_loop(0, _TP // 8, scan_blk, (nd0, ni0, nm0))
    ni_ref[...] = ni


def _gather_body(ni_ref, w2_ref, out_ref):
    iota_ex = (jax.lax.broadcasted_iota(jnp.int32, (_P, _WS), 1)
               - jax.lax.broadcasted_iota(jnp.int32, (_P, _WS), 0))
    iota_fy = (jax.lax.broadcasted_iota(jnp.int32, (2 * _WS, 2 * _P), 0)
               - jax.lax.broadcasted_iota(jnp.int32, (2 * _WS, 2 * _P), 1))
    for bc in range(_BC):
        wv = w2_ref[bc]
        for s in range(_N):
            t = ni_ref[s, bc]
            ok = t >= 0
            tt = jnp.maximum(t, 0)
            i = tt // _WS
            j = tt - i * _WS
            ei = (iota_ex == i).astype(jnp.float32)
            fj = (iota_fy == 2 * j).astype(jnp.float32)
            tmp = jnp.dot(wv, fj, preferred_element_type=jnp.float32,
                          precision=jax.lax.Precision.HIGHEST)
            patch = jnp.dot(ei, tmp, preferred_element_type=jnp.float32,
                            precision=jax.lax.Precision.HIGHEST)
            out_ref[bc, s] = jnp.where(ok, patch, 0.0)


def kernel(data, reference_index):
    ridx = reference_index.astype(jnp.int32)
    B, C = data.shape[0], data.shape[1]
    rx, ry = ridx[0], ridx[1]

    window = jax.lax.dynamic_slice(
        data, (0, 0, rx - _WS // 2, ry - _WS // 2, 0), (B, C, _WS, _WS, 2))
    rpatch = jax.lax.dynamic_slice(
        data, (0, 0, rx, ry, 0), (B, C, _P, _P, 2))

    wr = window[..., 0].reshape(_BC, _WS, _WS)
    wi = window[..., 1].reshape(_BC, _WS, _WS)
    wr = jnp.pad(wr, ((0, 0), (0, 8), (0, 0)))
    wi = jnp.pad(wi, ((0, 0), (0, 8), (0, 0)))
    rwr = rpatch[..., 0].reshape(_BC, _P, _P)
    rwi = rpatch[..., 1].reshape(_BC, _P, _P)

    dist = pl.pallas_call(
        _dist_body,
        grid=(),
        in_specs=[
            pl.BlockSpec(memory_space=pltpu.SMEM),
            pl.BlockSpec(memory_space=pltpu.VMEM),
            pl.BlockSpec(memory_space=pltpu.VMEM),
            pl.BlockSpec(memory_space=pltpu.VMEM),
            pl.BlockSpec(memory_space=pltpu.VMEM),
        ],
        out_specs=pl.BlockSpec(memory_space=pltpu.VMEM),
        out_shape=jax.ShapeDtypeStruct((_BC, 56, _WS), jnp.float32),
    )(ridx, wr, wi, rwr, rwi)

    distt = dist.reshape(_BC, _TP).T.reshape(_TP // 8, 8, _BC)

    ni = pl.pallas_call(
        _scan_body,
        grid=(),
        in_specs=[pl.BlockSpec(memory_space=pltpu.VMEM)],
        out_specs=pl.BlockSpec(memory_space=pltpu.VMEM),
        out_shape=jax.ShapeDtypeStruct((_N, _BC), jnp.int32),
    )(distt)

    w2 = window.reshape(_BC, _WS, 2 * _WS)
    patches = pl.pallas_call(
        _gather_body,
        grid=(),
        in_specs=[
            pl.BlockSpec(memory_space=pltpu.SMEM),
            pl.BlockSpec(memory_space=pltpu.VMEM),
        ],
        out_specs=pl.BlockSpec(memory_space=pltpu.VMEM),
        out_shape=jax.ShapeDtypeStruct((_BC, _N, _P, 2 * _P), jnp.float32),
    )(ni, w2)

    patches = patches.reshape(B, C, _N, _P * _P, 2)
    return jax.lax.complex(patches[..., 0], patches[..., 1])
